# baseline (speedup 1.0000x reference)
import jax
import jax.numpy as jnp
from jax import lax
from jax.experimental import pallas as pl
from jax.experimental.pallas import tpu as pltpu
from jax.experimental.pallas import tpu_sc as plsc

N = 10000
E = 320000
DF = 128
DH = 32
DO = 16

NC = 2
NS = 16
NW = NC * NS
CHUNK = 80
NCH = 125
EPW = NCH * CHUNK
RPT = N // NS

_SC_MESH = plsc.VectorSubcoreMesh(core_axis_name="c", subcore_axis_name="s")


def _deg_body(ei_hbm, out_hbm, dst_v, hist, sem):
    c = lax.axis_index("c")
    s = lax.axis_index("s")
    wid = s * NC + c

    zeros16 = jnp.zeros((16,), jnp.float32)

    def zero_body(i, carry):
        hist[pl.ds(i * 16, 16)] = zeros16
        return carry

    lax.fori_loop(0, N // 16, zero_body, 0)

    pltpu.sync_copy(ei_hbm.at[1, wid], dst_v)

    ones16 = jnp.ones((16,), jnp.float32)

    def chunk_body(j, carry):
        def vec_body(k, carry2):
            idx = dst_v[j, pl.ds(k * 16, 16)]
            plsc.addupdate_scatter(hist, [idx], ones16)
            return carry2

        return lax.fori_loop(0, CHUNK // 16, vec_body, carry)

    lax.fori_loop(0, NCH, chunk_body, 0)

    pltpu.sync_copy(hist, out_hbm.at[wid])


_deg_call = pl.kernel(
    _deg_body,
    out_type=jax.ShapeDtypeStruct((NW, N), jnp.float32),
    mesh=_SC_MESH,
    compiler_params=pltpu.CompilerParams(needs_layout_passes=False),
    scratch_types=[
        pltpu.VMEM((NCH, CHUNK), jnp.int32),
        pltpu.VMEM((N,), jnp.float32),
        pltpu.SemaphoreType.DMA,
    ],
)


def _make_agg(width):
    def body(hs_hbm, ei_hbm, out_hbm, src_v, dst_v, gbuf, zbuf, acc, gsem, ssem):
        c = lax.axis_index("c")
        s = lax.axis_index("s")
        wid = s * NC + c

        pltpu.sync_copy(ei_hbm.at[0, wid], src_v)
        pltpu.sync_copy(ei_hbm.at[1, wid], dst_v)

        zeros16 = jnp.zeros((16,), jnp.float32)

        def zrow(i, carry):
            def zcol(k, carry2):
                zbuf[i, pl.ds(k * 16, 16)] = zeros16
                return carry2

            return lax.fori_loop(0, width // 16, zcol, carry)

        lax.fori_loop(0, RPT, zrow, 0)
        pltpu.sync_copy(zbuf, acc.at[pl.ds(s * RPT, RPT)])
        plsc.subcore_barrier()

        for p in range(3):
            pltpu.async_copy(hs_hbm.at[src_v.at[p]], gbuf.at[p], gsem.at[p])

        def chunk_body(j, carry):
            b = lax.rem(j, 4)
            pltpu.make_async_copy(
                hs_hbm.at[src_v.at[j]], gbuf.at[b], gsem.at[b]
            ).wait()
            pltpu.async_copy(
                gbuf.at[b], acc.at[dst_v.at[j]], ssem.at[b], add=True
            )
            nxt = j + 3

            @pl.when(nxt < NCH)
            def _start():
                nb = lax.rem(nxt, 4)

                @pl.when(j >= 1)
                def _drain():
                    pltpu.make_async_copy(
                        gbuf.at[nb], acc.at[dst_v.at[j - 1]], ssem.at[nb]
                    ).wait()

                pltpu.async_copy(
                    hs_hbm.at[src_v.at[nxt]], gbuf.at[nb], gsem.at[nb]
                )

            return carry

        lax.fori_loop(0, NCH, chunk_body, 0)

        def drain_body(k, carry):
            j = NCH - 4 + k
            b = lax.rem(j, 4)
            pltpu.make_async_copy(
                gbuf.at[b], acc.at[dst_v.at[j]], ssem.at[b]
            ).wait()
            return carry

        lax.fori_loop(0, 4, drain_body, 0)
        plsc.subcore_barrier()

        pltpu.sync_copy(
            acc.at[pl.ds(s * RPT, RPT)], out_hbm.at[c, pl.ds(s * RPT, RPT)]
        )

    return pl.kernel(
        body,
        out_type=jax.ShapeDtypeStruct((NC, N, width), jnp.float32),
        mesh=_SC_MESH,
        compiler_params=pltpu.CompilerParams(use_tc_tiling_on_sc=False),
        scratch_types=[
            pltpu.VMEM((NCH, CHUNK), jnp.int32),
            pltpu.VMEM((NCH, CHUNK), jnp.int32),
            pltpu.VMEM((4, CHUNK, width), jnp.float32),
            pltpu.VMEM((RPT, width), jnp.float32),
            pltpu.VMEM_SHARED((N, width), jnp.float32),
            pltpu.SemaphoreType.DMA((4,)),
            pltpu.SemaphoreType.DMA((4,)),
        ],
    )


_agg_h = _make_agg(DH)
_agg_o = _make_agg(DO)


_BR = 400
_GRID = N // _BR


def _mm1_body(x_ref, w1_ref, h_ref):
    h_ref[...] = jnp.dot(
        x_ref[...], w1_ref[...], preferred_element_type=jnp.float32
    )


_mm1_call = pl.pallas_call(
    _mm1_body,
    grid=(_GRID,),
    in_specs=[
        pl.BlockSpec((_BR, DF), lambda i: (i, 0)),
        pl.BlockSpec((DF, DH), lambda i: (0, 0)),
    ],
    out_specs=pl.BlockSpec((_BR, DH), lambda i: (i, 0)),
    out_shape=jax.ShapeDtypeStruct((N, DH), jnp.float32),
)


def _scale_body(degp_ref, h_ref, hs_ref, dinv_ref):
    deg = jnp.sum(degp_ref[...], axis=0) + 1.0
    dinv = lax.rsqrt(deg)
    hs_ref[...] = h_ref[...] * dinv[:, None]
    dinv_ref[...] = dinv[:, None]


_scale_call = pl.pallas_call(
    _scale_body,
    grid=(1,),
    in_specs=[
        pl.BlockSpec((NW, N), lambda i: (0, 0)),
        pl.BlockSpec((N, DH), lambda i: (0, 0)),
    ],
    out_specs=[
        pl.BlockSpec((N, DH), lambda i: (0, 0)),
        pl.BlockSpec((N, 1), lambda i: (0, 0)),
    ],
    out_shape=[
        jax.ShapeDtypeStruct((N, DH), jnp.float32),
        jax.ShapeDtypeStruct((N, 1), jnp.float32),
    ],
)


def _l2_body(acc_ref, hs_ref, dinv_ref, wmu_ref, out_ref):
    dinv = dinv_ref[...]
    z = jnp.maximum((acc_ref[0] + acc_ref[1] + hs_ref[...]) * dinv, 0.0)
    h2 = jnp.dot(z, wmu_ref[...], preferred_element_type=jnp.float32)
    out_ref[...] = h2 * dinv


_l2_call = pl.pallas_call(
    _l2_body,
    grid=(_GRID,),
    in_specs=[
        pl.BlockSpec((NC, _BR, DH), lambda i: (0, i, 0)),
        pl.BlockSpec((_BR, DH), lambda i: (i, 0)),
        pl.BlockSpec((_BR, 1), lambda i: (i, 0)),
        pl.BlockSpec((DH, DO), lambda i: (0, 0)),
    ],
    out_specs=pl.BlockSpec((_BR, DO), lambda i: (i, 0)),
    out_shape=jax.ShapeDtypeStruct((N, DO), jnp.float32),
)


def _out_body(acc_ref, hs_ref, dinv_ref, out_ref):
    out_ref[...] = (acc_ref[0] + acc_ref[1] + hs_ref[...]) * dinv_ref[...]


_out_call = pl.pallas_call(
    _out_body,
    grid=(_GRID,),
    in_specs=[
        pl.BlockSpec((NC, _BR, DO), lambda i: (0, i, 0)),
        pl.BlockSpec((_BR, DO), lambda i: (i, 0)),
        pl.BlockSpec((_BR, 1), lambda i: (i, 0)),
    ],
    out_specs=pl.BlockSpec((_BR, DO), lambda i: (i, 0)),
    out_shape=jax.ShapeDtypeStruct((N, DO), jnp.float32),
)


def kernel(x, edge_index, W1, W_mu):
    ei = edge_index.astype(jnp.int32).reshape(2, NW, NCH, CHUNK)

    degp = _deg_call(ei)
    h1 = _mm1_call(x, W1)
    hs1, dinv = _scale_call(degp, h1)

    acc1 = _agg_h(hs1, ei)
    hs2 = _l2_call(acc1, hs1, dinv, W_mu)
    acc2 = _agg_o(hs2, ei)
    return _out_call(acc2, hs2, dinv)


# speedup vs baseline: 1.2425x; 1.2425x over previous
import jax
import jax.numpy as jnp
from jax import lax
from jax.experimental import pallas as pl
from jax.experimental.pallas import tpu as pltpu
from jax.experimental.pallas import tpu_sc as plsc

N = 10000
E = 320000
DF = 128
DH = 32
DO = 16

NC = 2
NS = 16
NW = NC * NS
CHUNK = 125
NCH = 80
EPW = NCH * CHUNK
RPT = N // NS

_SC_MESH = plsc.VectorSubcoreMesh(core_axis_name="c", subcore_axis_name="s")


def _deg_body(ei_hbm, out_hbm, dst_v, hist, sem):
    c = lax.axis_index("c")
    s = lax.axis_index("s")
    wid = s * NC + c

    zeros16 = jnp.zeros((16,), jnp.float32)

    def zero_body(i, carry):
        hist[pl.ds(i * 16, 16)] = zeros16
        return carry

    lax.fori_loop(0, N // 16, zero_body, 0)

    pltpu.sync_copy(ei_hbm.at[1, wid], dst_v)

    ones16 = jnp.ones((16,), jnp.float32)
    tail_mask = lax.iota(jnp.int32, 16) >= 3

    def chunk_body(j, carry):
        def vec_body(k, carry2):
            idx = dst_v[j, pl.ds(k * 16, 16)]
            plsc.addupdate_scatter(hist, [idx], ones16)
            return carry2

        lax.fori_loop(0, 7, vec_body, 0)
        idx_t = dst_v[j, pl.ds(CHUNK - 16, 16)]
        plsc.addupdate_scatter(hist, [idx_t], ones16, mask=tail_mask)
        return carry

    lax.fori_loop(0, NCH, chunk_body, 0)

    pltpu.sync_copy(hist, out_hbm.at[wid])


_deg_call = pl.kernel(
    _deg_body,
    out_type=jax.ShapeDtypeStruct((NW, N), jnp.float32),
    mesh=_SC_MESH,
    compiler_params=pltpu.CompilerParams(needs_layout_passes=False),
    scratch_types=[
        pltpu.VMEM((NCH, CHUNK), jnp.int32),
        pltpu.VMEM((N,), jnp.float32),
        pltpu.SemaphoreType.DMA,
    ],
)


def _make_agg(width):
    def body(hs_hbm, ei_hbm, out_hbm, src_v, dst_v, gbuf, zbuf, acc, gsem, ssem):
        c = lax.axis_index("c")
        s = lax.axis_index("s")
        wid = s * NC + c

        pltpu.sync_copy(ei_hbm.at[0, wid], src_v)
        pltpu.sync_copy(ei_hbm.at[1, wid], dst_v)

        zeros16 = jnp.zeros((16,), jnp.float32)

        def zrow(i, carry):
            def zcol(k, carry2):
                zbuf[i, pl.ds(k * 16, 16)] = zeros16
                return carry2

            return lax.fori_loop(0, width // 16, zcol, carry)

        lax.fori_loop(0, RPT, zrow, 0)
        pltpu.sync_copy(zbuf, acc.at[pl.ds(s * RPT, RPT)])
        plsc.subcore_barrier()

        for p in range(3):
            pltpu.async_copy(hs_hbm.at[src_v.at[p]], gbuf.at[p], gsem.at[p])

        def chunk_body(j, carry):
            b = lax.rem(j, 4)
            pltpu.make_async_copy(
                hs_hbm.at[src_v.at[j]], gbuf.at[b], gsem.at[b]
            ).wait()
            pltpu.async_copy(
                gbuf.at[b], acc.at[dst_v.at[j]], ssem.at[b], add=True
            )
            nxt = j + 3

            @pl.when(nxt < NCH)
            def _start():
                nb = lax.rem(nxt, 4)

                @pl.when(j >= 1)
                def _drain():
                    pltpu.make_async_copy(
                        gbuf.at[nb], acc.at[dst_v.at[j - 1]], ssem.at[nb]
                    ).wait()

                pltpu.async_copy(
                    hs_hbm.at[src_v.at[nxt]], gbuf.at[nb], gsem.at[nb]
                )

            return carry

        lax.fori_loop(0, NCH, chunk_body, 0)

        def drain_body(k, carry):
            j = NCH - 4 + k
            b = lax.rem(j, 4)
            pltpu.make_async_copy(
                gbuf.at[b], acc.at[dst_v.at[j]], ssem.at[b]
            ).wait()
            return carry

        lax.fori_loop(0, 4, drain_body, 0)
        plsc.subcore_barrier()

        pltpu.sync_copy(
            acc.at[pl.ds(s * RPT, RPT)], out_hbm.at[c, pl.ds(s * RPT, RPT)]
        )

    return pl.kernel(
        body,
        out_type=jax.ShapeDtypeStruct((NC, N, width), jnp.float32),
        mesh=_SC_MESH,
        compiler_params=pltpu.CompilerParams(use_tc_tiling_on_sc=False),
        scratch_types=[
            pltpu.VMEM((NCH, CHUNK), jnp.int32),
            pltpu.VMEM((NCH, CHUNK), jnp.int32),
            pltpu.VMEM((4, CHUNK, width), jnp.float32),
            pltpu.VMEM((RPT, width), jnp.float32),
            pltpu.VMEM_SHARED((N, width), jnp.float32),
            pltpu.SemaphoreType.DMA((4,)),
            pltpu.SemaphoreType.DMA((4,)),
        ],
    )


_agg_h = _make_agg(DH)
_agg_o = _make_agg(DO)


_BR = 2000
_GRID = N // _BR


def _mm1_body(x_ref, w1_ref, h_ref):
    h_ref[...] = jnp.dot(
        x_ref[...], w1_ref[...], preferred_element_type=jnp.float32
    )


_mm1_call = pl.pallas_call(
    _mm1_body,
    grid=(_GRID,),
    in_specs=[
        pl.BlockSpec((_BR, DF), lambda i: (i, 0)),
        pl.BlockSpec((DF, DH), lambda i: (0, 0)),
    ],
    out_specs=pl.BlockSpec((_BR, DH), lambda i: (i, 0)),
    out_shape=jax.ShapeDtypeStruct((N, DH), jnp.float32),
)


def _dinv_body(degp_ref, dinv_ref):
    deg = jnp.sum(degp_ref[...], axis=0) + 1.0
    dinv_ref[...] = lax.rsqrt(deg)[:, None]


_dinv_call = pl.pallas_call(
    _dinv_body,
    grid=(1,),
    in_specs=[pl.BlockSpec((NW, N), lambda i: (0, 0))],
    out_specs=pl.BlockSpec((N, 1), lambda i: (0, 0)),
    out_shape=jax.ShapeDtypeStruct((N, 1), jnp.float32),
)


def _scale_body(h_ref, dinv_ref, hs_ref):
    hs_ref[...] = h_ref[...] * dinv_ref[...]


_scale_call = pl.pallas_call(
    _scale_body,
    grid=(_GRID,),
    in_specs=[
        pl.BlockSpec((_BR, DH), lambda i: (i, 0)),
        pl.BlockSpec((_BR, 1), lambda i: (i, 0)),
    ],
    out_specs=pl.BlockSpec((_BR, DH), lambda i: (i, 0)),
    out_shape=jax.ShapeDtypeStruct((N, DH), jnp.float32),
)


def _l2_body(acc_ref, hs_ref, dinv_ref, wmu_ref, out_ref):
    dinv = dinv_ref[...]
    z = jnp.maximum((acc_ref[0] + acc_ref[1] + hs_ref[...]) * dinv, 0.0)
    h2 = jnp.dot(z, wmu_ref[...], preferred_element_type=jnp.float32)
    out_ref[...] = h2 * dinv


_l2_call = pl.pallas_call(
    _l2_body,
    grid=(_GRID,),
    in_specs=[
        pl.BlockSpec((NC, _BR, DH), lambda i: (0, i, 0)),
        pl.BlockSpec((_BR, DH), lambda i: (i, 0)),
        pl.BlockSpec((_BR, 1), lambda i: (i, 0)),
        pl.BlockSpec((DH, DO), lambda i: (0, 0)),
    ],
    out_specs=pl.BlockSpec((_BR, DO), lambda i: (i, 0)),
    out_shape=jax.ShapeDtypeStruct((N, DO), jnp.float32),
)


def _out_body(acc_ref, hs_ref, dinv_ref, out_ref):
    out_ref[...] = (acc_ref[0] + acc_ref[1] + hs_ref[...]) * dinv_ref[...]


_out_call = pl.pallas_call(
    _out_body,
    grid=(_GRID,),
    in_specs=[
        pl.BlockSpec((NC, _BR, DO), lambda i: (0, i, 0)),
        pl.BlockSpec((_BR, DO), lambda i: (i, 0)),
        pl.BlockSpec((_BR, 1), lambda i: (i, 0)),
    ],
    out_specs=pl.BlockSpec((_BR, DO), lambda i: (i, 0)),
    out_shape=jax.ShapeDtypeStruct((N, DO), jnp.float32),
)


def kernel(x, edge_index, W1, W_mu):
    ei = edge_index.astype(jnp.int32).reshape(2, NW, NCH, CHUNK)

    degp = _deg_call(ei)
    h1 = _mm1_call(x, W1)
    dinv = _dinv_call(degp)
    hs1 = _scale_call(h1, dinv)

    acc1 = _agg_h(hs1, ei)
    hs2 = _l2_call(acc1, hs1, dinv, W_mu)
    acc2 = _agg_o(hs2, ei)
    return _out_call(acc2, hs2, dinv)


# speedup vs baseline: 1.3543x; 1.0899x over previous
import jax
import jax.numpy as jnp
from jax import lax
from jax.experimental import pallas as pl
from jax.experimental.pallas import tpu as pltpu
from jax.experimental.pallas import tpu_sc as plsc

N = 10000
E = 320000
DF = 128
DH = 32
DO = 16

NC = 2
NS = 16
NW = NC * NS
CHUNK = 125
NCH = 80
EPW = NCH * CHUNK
RPT = N // NS
NBUF = 8

_SC_MESH = plsc.VectorSubcoreMesh(core_axis_name="c", subcore_axis_name="s")


def _deg_body(ei_hbm, out_hbm, dst_v, hist, sem):
    c = lax.axis_index("c")
    s = lax.axis_index("s")
    wid = s * NC + c

    zeros16 = jnp.zeros((16,), jnp.float32)

    def zero_body(i, carry):
        hist[pl.ds(i * 16, 16)] = zeros16
        return carry

    lax.fori_loop(0, N // 16, zero_body, 0)

    pltpu.sync_copy(ei_hbm.at[1, wid], dst_v)

    ones16 = jnp.ones((16,), jnp.float32)
    tail_mask = lax.iota(jnp.int32, 16) >= 3

    def chunk_body(j, carry):
        for k in range(7):
            idx = dst_v[j, pl.ds(k * 16, 16)]
            plsc.addupdate_scatter(hist, [idx], ones16)
        idx_t = dst_v[j, pl.ds(CHUNK - 16, 16)]
        plsc.addupdate_scatter(hist, [idx_t], ones16, mask=tail_mask)
        return carry

    lax.fori_loop(0, NCH, chunk_body, 0)

    pltpu.sync_copy(hist, out_hbm.at[wid])


_deg_call = pl.kernel(
    _deg_body,
    out_type=jax.ShapeDtypeStruct((NW, N), jnp.float32),
    mesh=_SC_MESH,
    compiler_params=pltpu.CompilerParams(needs_layout_passes=False),
    scratch_types=[
        pltpu.VMEM((NCH, CHUNK), jnp.int32),
        pltpu.VMEM((N,), jnp.float32),
        pltpu.SemaphoreType.DMA,
    ],
)


def _make_agg(width):
    def body(hs_hbm, ei_hbm, out_hbm, src_v, dst_v, gbuf, zbuf, acc, gsem, ssem):
        c = lax.axis_index("c")
        s = lax.axis_index("s")
        wid = s * NC + c

        pltpu.sync_copy(ei_hbm.at[0, wid], src_v)
        pltpu.sync_copy(ei_hbm.at[1, wid], dst_v)

        zeros16 = jnp.zeros((16,), jnp.float32)

        def zrow(i, carry):
            def zcol(k, carry2):
                zbuf[i, pl.ds(k * 16, 16)] = zeros16
                return carry2

            return lax.fori_loop(0, width // 16, zcol, carry)

        lax.fori_loop(0, RPT, zrow, 0)
        pltpu.sync_copy(zbuf, acc.at[pl.ds(s * RPT, RPT)])
        plsc.subcore_barrier()

        for p in range(NBUF - 1):
            pltpu.async_copy(hs_hbm.at[src_v.at[p]], gbuf.at[p], gsem.at[p])

        def chunk_body(j, carry):
            b = lax.rem(j, NBUF)
            pltpu.make_async_copy(
                hs_hbm.at[src_v.at[j]], gbuf.at[b], gsem.at[b]
            ).wait()
            pltpu.async_copy(
                gbuf.at[b], acc.at[dst_v.at[j]], ssem.at[b], add=True
            )
            nxt = j + NBUF - 1

            @pl.when(nxt < NCH)
            def _start():
                nb = lax.rem(nxt, NBUF)

                @pl.when(j >= 1)
                def _drain():
                    pltpu.make_async_copy(
                        gbuf.at[nb], acc.at[dst_v.at[j - 1]], ssem.at[nb]
                    ).wait()

                pltpu.async_copy(
                    hs_hbm.at[src_v.at[nxt]], gbuf.at[nb], gsem.at[nb]
                )

            return carry

        lax.fori_loop(0, NCH, chunk_body, 0)

        def drain_body(k, carry):
            j = NCH - NBUF + k
            b = lax.rem(j, NBUF)
            pltpu.make_async_copy(
                gbuf.at[b], acc.at[dst_v.at[j]], ssem.at[b]
            ).wait()
            return carry

        lax.fori_loop(0, NBUF, drain_body, 0)
        plsc.subcore_barrier()

        pltpu.sync_copy(
            acc.at[pl.ds(s * RPT, RPT)], out_hbm.at[c, pl.ds(s * RPT, RPT)]
        )

    return pl.kernel(
        body,
        out_type=jax.ShapeDtypeStruct((NC, N, width), jnp.float32),
        mesh=_SC_MESH,
        compiler_params=pltpu.CompilerParams(use_tc_tiling_on_sc=False),
        scratch_types=[
            pltpu.VMEM((NCH, CHUNK), jnp.int32),
            pltpu.VMEM((NCH, CHUNK), jnp.int32),
            pltpu.VMEM((NBUF, CHUNK, width), jnp.float32),
            pltpu.VMEM((RPT, width), jnp.float32),
            pltpu.VMEM_SHARED((N, width), jnp.float32),
            pltpu.SemaphoreType.DMA((NBUF,)),
            pltpu.SemaphoreType.DMA((NBUF,)),
        ],
    )


_agg_h = _make_agg(DH)
_agg_o = _make_agg(DO)


_BR = 2000
_GRID = N // _BR


def _dinv_body(degp_ref, dinv_ref):
    deg = jnp.sum(degp_ref[...], axis=0) + 1.0
    dinv = lax.rsqrt(deg)
    dinv_ref[...] = jnp.broadcast_to(dinv[:, None], (N, DH))


_dinv_call = pl.pallas_call(
    _dinv_body,
    grid=(1,),
    in_specs=[pl.BlockSpec((NW, N), lambda i: (0, 0))],
    out_specs=pl.BlockSpec((N, DH), lambda i: (0, 0)),
    out_shape=jax.ShapeDtypeStruct((N, DH), jnp.float32),
)


def _mm1_body(x_ref, w1_ref, dinv_ref, hs_ref):
    h = jnp.dot(x_ref[...], w1_ref[...], preferred_element_type=jnp.float32)
    hs_ref[...] = h * dinv_ref[...]


_mm1_call = pl.pallas_call(
    _mm1_body,
    grid=(_GRID,),
    in_specs=[
        pl.BlockSpec((_BR, DF), lambda i: (i, 0)),
        pl.BlockSpec((DF, DH), lambda i: (0, 0)),
        pl.BlockSpec((_BR, DH), lambda i: (i, 0)),
    ],
    out_specs=pl.BlockSpec((_BR, DH), lambda i: (i, 0)),
    out_shape=jax.ShapeDtypeStruct((N, DH), jnp.float32),
)


def _l2_body(acc_ref, hs_ref, dinv_ref, wmu_ref, out_ref):
    dinv = dinv_ref[...]
    z = jnp.maximum((acc_ref[0] + acc_ref[1] + hs_ref[...]) * dinv, 0.0)
    h2 = jnp.dot(z, wmu_ref[...], preferred_element_type=jnp.float32)
    out_ref[...] = h2 * dinv[:, :DO]


_l2_call = pl.pallas_call(
    _l2_body,
    grid=(_GRID,),
    in_specs=[
        pl.BlockSpec((NC, _BR, DH), lambda i: (0, i, 0)),
        pl.BlockSpec((_BR, DH), lambda i: (i, 0)),
        pl.BlockSpec((_BR, DH), lambda i: (i, 0)),
        pl.BlockSpec((DH, DO), lambda i: (0, 0)),
    ],
    out_specs=pl.BlockSpec((_BR, DO), lambda i: (i, 0)),
    out_shape=jax.ShapeDtypeStruct((N, DO), jnp.float32),
)


def _out_body(acc_ref, hs_ref, dinv_ref, out_ref):
    out_ref[...] = (
        (acc_ref[0] + acc_ref[1] + hs_ref[...]) * dinv_ref[:, :DO]
    )


_out_call = pl.pallas_call(
    _out_body,
    grid=(_GRID,),
    in_specs=[
        pl.BlockSpec((NC, _BR, DO), lambda i: (0, i, 0)),
        pl.BlockSpec((_BR, DO), lambda i: (i, 0)),
        pl.BlockSpec((_BR, DH), lambda i: (i, 0)),
    ],
    out_specs=pl.BlockSpec((_BR, DO), lambda i: (i, 0)),
    out_shape=jax.ShapeDtypeStruct((N, DO), jnp.float32),
)


def kernel(x, edge_index, W1, W_mu):
    ei = edge_index.astype(jnp.int32).reshape(2, NW, NCH, CHUNK)

    degp = _deg_call(ei)
    dinv = _dinv_call(degp)
    hs1 = _mm1_call(x, W1, dinv)

    acc1 = _agg_h(hs1, ei)
    hs2 = _l2_call(acc1, hs1, dinv, W_mu)
    acc2 = _agg_o(hs2, ei)
    return _out_call(acc2, hs2, dinv)


# speedup vs baseline: 1.3634x; 1.0067x over previous
import jax
import jax.numpy as jnp
from jax import lax
from jax.experimental import pallas as pl
from jax.experimental.pallas import tpu as pltpu
from jax.experimental.pallas import tpu_sc as plsc

N = 10000
E = 320000
DF = 128
DH = 32
DO = 16

NC = 2
NS = 16
NW = NC * NS
CHUNK = 125
NCH = 80
EPW = NCH * CHUNK
RPT = N // NS
NBUF = 8

_SC_MESH = plsc.VectorSubcoreMesh(core_axis_name="c", subcore_axis_name="s")


def _deg_body(ei_hbm, out_hbm, dst_v, hist, sem):
    c = lax.axis_index("c")
    s = lax.axis_index("s")
    wid = s * NC + c

    zeros16 = jnp.zeros((16,), jnp.float32)

    def zero_body(i, carry):
        hist[pl.ds(i * 16, 16)] = zeros16
        return carry

    lax.fori_loop(0, N // 16, zero_body, 0)

    pltpu.sync_copy(ei_hbm.at[1, wid], dst_v)

    ones16 = jnp.ones((16,), jnp.float32)
    tail_mask = lax.iota(jnp.int32, 16) >= 3

    def chunk_body(j, carry):
        for k in range(7):
            idx = dst_v[j, pl.ds(k * 16, 16)]
            plsc.addupdate_scatter(hist, [idx], ones16)
        idx_t = dst_v[j, pl.ds(CHUNK - 16, 16)]
        plsc.addupdate_scatter(hist, [idx_t], ones16, mask=tail_mask)
        return carry

    lax.fori_loop(0, NCH, chunk_body, 0)

    pltpu.sync_copy(hist, out_hbm.at[wid])


_deg_call = pl.kernel(
    _deg_body,
    out_type=jax.ShapeDtypeStruct((NW, N), jnp.float32),
    mesh=_SC_MESH,
    compiler_params=pltpu.CompilerParams(needs_layout_passes=False),
    scratch_types=[
        pltpu.VMEM((NCH, CHUNK), jnp.int32),
        pltpu.VMEM((N,), jnp.float32),
        pltpu.SemaphoreType.DMA,
    ],
)


def _make_agg(width):
    def body(hs_hbm, ei_hbm, out_hbm, src_v, dst_v, gbuf, zbuf, acc, gsem, ssem):
        c = lax.axis_index("c")
        s = lax.axis_index("s")
        wid = s * NC + c

        pltpu.sync_copy(ei_hbm.at[0, wid], src_v)
        pltpu.sync_copy(ei_hbm.at[1, wid], dst_v)

        zeros16 = jnp.zeros((16,), jnp.float32)

        def zrow(i, carry):
            def zcol(k, carry2):
                zbuf[i, pl.ds(k * 16, 16)] = zeros16
                return carry2

            return lax.fori_loop(0, width // 16, zcol, carry)

        lax.fori_loop(0, RPT, zrow, 0)
        pltpu.sync_copy(zbuf, acc.at[pl.ds(s * RPT, RPT)])
        plsc.subcore_barrier()

        for p in range(NBUF - 1):
            pltpu.async_copy(hs_hbm.at[src_v.at[p]], gbuf.at[p], gsem.at[p])

        def chunk_body(j, carry):
            b = lax.rem(j, NBUF)
            pltpu.make_async_copy(
                hs_hbm.at[src_v.at[j]], gbuf.at[b], gsem.at[b]
            ).wait()
            pltpu.async_copy(
                gbuf.at[b], acc.at[dst_v.at[j]], ssem.at[b], add=True
            )
            nxt = j + NBUF - 1

            @pl.when(nxt < NCH)
            def _start():
                nb = lax.rem(nxt, NBUF)

                @pl.when(j >= 1)
                def _drain():
                    pltpu.make_async_copy(
                        gbuf.at[nb], acc.at[dst_v.at[j - 1]], ssem.at[nb]
                    ).wait()

                pltpu.async_copy(
                    hs_hbm.at[src_v.at[nxt]], gbuf.at[nb], gsem.at[nb]
                )

            return carry

        lax.fori_loop(0, NCH, chunk_body, 0)

        def drain_body(k, carry):
            j = NCH - NBUF + k
            b = lax.rem(j, NBUF)
            pltpu.make_async_copy(
                gbuf.at[b], acc.at[dst_v.at[j]], ssem.at[b]
            ).wait()
            return carry

        lax.fori_loop(0, NBUF, drain_body, 0)
        plsc.subcore_barrier()

        pltpu.sync_copy(
            acc.at[pl.ds(s * RPT, RPT)], out_hbm.at[c, pl.ds(s * RPT, RPT)]
        )

    return pl.kernel(
        body,
        out_type=jax.ShapeDtypeStruct((NC, N, width), jnp.float32),
        mesh=_SC_MESH,
        compiler_params=pltpu.CompilerParams(use_tc_tiling_on_sc=False),
        scratch_types=[
            pltpu.VMEM((NCH, CHUNK), jnp.int32),
            pltpu.VMEM((NCH, CHUNK), jnp.int32),
            pltpu.VMEM((NBUF, CHUNK, width), jnp.float32),
            pltpu.VMEM((RPT, width), jnp.float32),
            pltpu.VMEM_SHARED((N, width), jnp.float32),
            pltpu.SemaphoreType.DMA((NBUF,)),
            pltpu.SemaphoreType.DMA((NBUF,)),
        ],
    )


_agg_h = _make_agg(DH)
_agg_o = _make_agg(DO)


_BR = 2000
_GRID = N // _BR


def _dinv_body(degp_ref, dinv_ref):
    deg = jnp.sum(degp_ref[...], axis=0) + 1.0
    dinv = lax.rsqrt(deg)
    dinv_ref[...] = jnp.broadcast_to(dinv[:, None], (N, DH))


_dinv_call = pl.pallas_call(
    _dinv_body,
    grid=(1,),
    in_specs=[pl.BlockSpec((NW, N), lambda i: (0, 0))],
    out_specs=pl.BlockSpec((N, DH), lambda i: (0, 0)),
    out_shape=jax.ShapeDtypeStruct((N, DH), jnp.float32),
)


def _mm1_body(x_ref, w1_ref, h_ref):
    h_ref[...] = jnp.dot(
        x_ref[...], w1_ref[...], preferred_element_type=jnp.float32
    )


_mm1_call = pl.pallas_call(
    _mm1_body,
    grid=(_GRID,),
    in_specs=[
        pl.BlockSpec((_BR, DF), lambda i: (i, 0)),
        pl.BlockSpec((DF, DH), lambda i: (0, 0)),
    ],
    out_specs=pl.BlockSpec((_BR, DH), lambda i: (i, 0)),
    out_shape=jax.ShapeDtypeStruct((N, DH), jnp.float32),
)


def _scale_body(h_ref, dinv_ref, hs_ref):
    hs_ref[...] = h_ref[...] * dinv_ref[...]


_scale_call = pl.pallas_call(
    _scale_body,
    grid=(_GRID,),
    in_specs=[
        pl.BlockSpec((_BR, DH), lambda i: (i, 0)),
        pl.BlockSpec((_BR, DH), lambda i: (i, 0)),
    ],
    out_specs=pl.BlockSpec((_BR, DH), lambda i: (i, 0)),
    out_shape=jax.ShapeDtypeStruct((N, DH), jnp.float32),
)


def _l2_body(acc_ref, hs_ref, dinv_ref, wmu_ref, out_ref):
    dinv = dinv_ref[...]
    z = jnp.maximum((acc_ref[0] + acc_ref[1] + hs_ref[...]) * dinv, 0.0)
    h2 = jnp.dot(z, wmu_ref[...], preferred_element_type=jnp.float32)
    out_ref[...] = h2 * dinv[:, :DO]


_l2_call = pl.pallas_call(
    _l2_body,
    grid=(_GRID,),
    in_specs=[
        pl.BlockSpec((NC, _BR, DH), lambda i: (0, i, 0)),
        pl.BlockSpec((_BR, DH), lambda i: (i, 0)),
        pl.BlockSpec((_BR, DH), lambda i: (i, 0)),
        pl.BlockSpec((DH, DO), lambda i: (0, 0)),
    ],
    out_specs=pl.BlockSpec((_BR, DO), lambda i: (i, 0)),
    out_shape=jax.ShapeDtypeStruct((N, DO), jnp.float32),
)


def _out_body(acc_ref, hs_ref, dinv_ref, out_ref):
    out_ref[...] = (
        (acc_ref[0] + acc_ref[1] + hs_ref[...]) * dinv_ref[:, :DO]
    )


_out_call = pl.pallas_call(
    _out_body,
    grid=(_GRID,),
    in_specs=[
        pl.BlockSpec((NC, _BR, DO), lambda i: (0, i, 0)),
        pl.BlockSpec((_BR, DO), lambda i: (i, 0)),
        pl.BlockSpec((_BR, DH), lambda i: (i, 0)),
    ],
    out_specs=pl.BlockSpec((_BR, DO), lambda i: (i, 0)),
    out_shape=jax.ShapeDtypeStruct((N, DO), jnp.float32),
)


def kernel(x, edge_index, W1, W_mu):
    ei = edge_index.astype(jnp.int32).reshape(2, NW, NCH, CHUNK)

    degp = _deg_call(ei)
    h1 = _mm1_call(x, W1)
    dinv = _dinv_call(degp)
    hs1 = _scale_call(h1, dinv)

    acc1 = _agg_h(hs1, ei)
    hs2 = _l2_call(acc1, hs1, dinv, W_mu)
    acc2 = _agg_o(hs2, ei)
    return _out_call(acc2, hs2, dinv)
